# Initial kernel scaffold; baseline (speedup 1.0000x reference)
import jax
import jax.numpy as jnp
from jax.experimental import pallas as pl


def kernel(x, edge_index, batch_idx, table, W1, att_src1, att_dst1, b1, W2, att_src2, att_dst2, b2, Wp, bp):
    raise NotImplementedError("write your pallas kernel here")


# speedup vs baseline: 30.9731x; 30.9731x over previous
import functools

import jax
import jax.numpy as jnp
from jax import lax
from jax.experimental import pallas as pl
from jax.experimental.pallas import tpu as pltpu
from jax.experimental.pallas import tpu_sc as plsc

N = 50000
E = 800000
NPAD = 50176
EPAD = 802816
NG = 64
F32 = jnp.float32
I32 = jnp.int32

NT = 3136
TE1 = EPAD // 32
TE2 = EPAD // 16

_mesh = lambda: plsc.VectorSubcoreMesh(core_axis_name="c", subcore_axis_name="s",
                                       num_cores=2)
_SC_PARAMS = pltpu.CompilerParams(use_tc_tiling_on_sc=False,
                                  needs_layout_passes=False)


def _leaky(x, s):
    return jnp.where(x >= 0, x, s * x)


def _dense_body(x_ref, w_ref, b_ref, o_ref):
    o_ref[...] = jnp.dot(x_ref[...], w_ref[...],
                         preferred_element_type=F32) + b_ref[...]


def k_dense(x, W, b, blk=3136):
    m = W.shape[1]
    return pl.pallas_call(
        _dense_body,
        grid=(NPAD // blk,),
        in_specs=[
            pl.BlockSpec((blk, x.shape[1]), lambda i: (i, 0)),
            pl.BlockSpec(W.shape, lambda i: (0, 0)),
            pl.BlockSpec((1, m), lambda i: (0, 0)),
        ],
        out_specs=pl.BlockSpec((blk, m), lambda i: (i, 0)),
        out_shape=jax.ShapeDtypeStruct((NPAD, m), F32),
    )(x, W, b.reshape(1, m))


def _layer_head_body(x_ref, w_ref, asrc_ref, adst_ref, h_ref, asad_ref):
    h = jnp.dot(x_ref[...], w_ref[...], preferred_element_type=F32)
    h_ref[...] = h
    a_s, a_d = asrc_ref[...], adst_ref[...]
    cols = []
    for hd in range(2):
        cols.append(jnp.sum(h[:, 64 * hd:64 * hd + 64] * a_s[hd:hd + 1, :],
                            axis=1))
    for hd in range(2):
        cols.append(jnp.sum(h[:, 64 * hd:64 * hd + 64] * a_d[hd:hd + 1, :],
                            axis=1))
    asad_ref[...] = jnp.stack(cols, axis=1)


def k_layer_head(x, W, a_src, a_dst, blk=3136):
    k = x.shape[1]
    return pl.pallas_call(
        _layer_head_body,
        grid=(NPAD // blk,),
        in_specs=[
            pl.BlockSpec((blk, k), lambda i: (i, 0)),
            pl.BlockSpec((k, 128), lambda i: (0, 0)),
            pl.BlockSpec((2, 64), lambda i: (0, 0)),
            pl.BlockSpec((2, 64), lambda i: (0, 0)),
        ],
        out_specs=[
            pl.BlockSpec((blk, 128), lambda i: (i, 0)),
            pl.BlockSpec((blk, 4), lambda i: (i, 0)),
        ],
        out_shape=[
            jax.ShapeDtypeStruct((NPAD, 128), F32),
            jax.ShapeDtypeStruct((NPAD, 4), F32),
        ],
    )(x, W, a_src, a_dst)


def _combine_body(acc_ref, h_ref, asad_ref, dp_ref, b_ref, o_ref):
    asad = asad_ref[...]
    dp = dp_ref[...]
    wself = jnp.exp(_leaky(asad[:, 0:2] + asad[:, 2:4], 0.2))
    denom = dp[:, 0:2] + dp[:, 2:4] + wself
    h = h_ref[...]
    acc = acc_ref[...]
    outs = []
    for hd in range(2):
        num = acc[:, 64 * hd:64 * hd + 64] + \
            wself[:, hd:hd + 1] * h[:, 64 * hd:64 * hd + 64]
        outs.append(num / (denom[:, hd:hd + 1] + 1e-16))
    out = jnp.concatenate(outs, axis=1) + b_ref[...]
    o_ref[...] = _leaky(out, 0.01)


def k_combine(accf, h, asad, dp_n, b, blk=3136):
    return pl.pallas_call(
        _combine_body,
        grid=(NPAD // blk,),
        in_specs=[
            pl.BlockSpec((blk, 128), lambda i: (i, 0)),
            pl.BlockSpec((blk, 128), lambda i: (i, 0)),
            pl.BlockSpec((blk, 4), lambda i: (i, 0)),
            pl.BlockSpec((blk, 4), lambda i: (i, 0)),
            pl.BlockSpec((1, 128), lambda i: (0, 0)),
        ],
        out_specs=pl.BlockSpec((blk, 128), lambda i: (i, 0)),
        out_shape=jax.ShapeDtypeStruct((NPAD, 128), F32),
    )(accf, h, asad, dp_n, b.reshape(1, 128))


def _pool_body(hp_ref, bidx_ref, z_ref, sums, cnts):
    i = pl.program_id(0)
    ng = pl.num_programs(0)

    @pl.when(i == 0)
    def _():
        sums[...] = jnp.zeros_like(sums)
        cnts[...] = jnp.zeros_like(cnts)

    bidx = bidx_ref[0]
    giota = lax.broadcasted_iota(I32, (NG, 1), 0)
    oh = (bidx == giota).astype(F32)
    hp = hp_ref[...]
    sums[...] += jnp.dot(oh, hp, preferred_element_type=F32)
    cnts[...] += jnp.dot(oh, jnp.ones_like(hp), preferred_element_type=F32)

    @pl.when(i == ng - 1)
    def _():
        z_ref[...] = sums[...] / jnp.maximum(cnts[...], 1.0)


def k_pool(hp, bidx, blk=3136):
    return pl.pallas_call(
        _pool_body,
        grid=(NPAD // blk,),
        in_specs=[
            pl.BlockSpec((blk, 64), lambda i: (i, 0)),
            pl.BlockSpec((1, 1, blk), lambda i: (i, 0, 0)),
        ],
        out_specs=pl.BlockSpec((NG, 64), lambda i: (0, 0)),
        out_shape=jax.ShapeDtypeStruct((NG, 64), F32),
        scratch_shapes=[pltpu.VMEM((NG, 64), F32), pltpu.VMEM((NG, 64), F32)],
    )(hp, bidx.reshape(NPAD // blk, 1, blk))


def k_embed(table, idx):
    rows_per_tile = NPAD // 32

    @functools.partial(
        pl.kernel, mesh=_mesh(), compiler_params=_SC_PARAMS,
        out_type=jax.ShapeDtypeStruct((NPAD, 64), F32),
        scratch_types=[
            pltpu.VMEM((rows_per_tile,), I32),
            pltpu.VMEM((112, 64), F32),
            pltpu.SemaphoreType.DMA,
        ],
    )
    def _k(table_hbm, idx_hbm, out_hbm, idx_v, rows_v, sem):
        wid = lax.axis_index("c") * 16 + lax.axis_index("s")
        base = wid * rows_per_tile
        pltpu.sync_copy(idx_hbm.at[pl.ds(base, rows_per_tile)], idx_v)

        @pl.loop(0, 14)
        def _(ci):
            off = ci * 112
            pltpu.async_copy(table_hbm.at[idx_v.at[pl.ds(off, 112)]],
                             rows_v, sem).wait()
            pltpu.sync_copy(rows_v, out_hbm.at[pl.ds(base + off, 112)])

    return _k(table, idx)


def k_pass1(asad_t, src, dst):
    @functools.partial(
        pl.kernel, mesh=_mesh(), compiler_params=_SC_PARAMS,
        out_type=[jax.ShapeDtypeStruct((2 * EPAD,), F32),
                  jax.ShapeDtypeStruct((4 * NPAD,), F32)],
        scratch_types=[
            pltpu.VMEM((NPAD,), F32),
            pltpu.VMEM((NPAD,), F32),
            pltpu.VMEM((128,), I32),
            pltpu.VMEM((128,), I32),
            pltpu.VMEM((128,), F32),
            pltpu.VMEM((NT,), F32),
            pltpu.VMEM_SHARED((NPAD,), F32),
            pltpu.SemaphoreType.DMA,
        ],
    )
    def _k(asad_hbm, src_hbm, dst_hbm, w_hbm, dp_hbm,
           as_v, ad_v, srcv, dstv, wbuf, zbuf, dsh, sem):
        core = lax.axis_index("c")
        sub = lax.axis_index("s")
        wid = core * 16 + sub

        @pl.loop(0, NT, step=16)
        def _(i):
            zbuf[pl.ds(i, 16)] = jnp.zeros((16,), F32)

        for head in range(2):
            pltpu.sync_copy(zbuf, dsh.at[pl.ds(sub * NT, NT)])
            pltpu.sync_copy(asad_hbm.at[pl.ds(head * NPAD, NPAD)], as_v)
            pltpu.sync_copy(asad_hbm.at[pl.ds((2 + head) * NPAD, NPAD)], ad_v)
            plsc.subcore_barrier()

            @pl.loop(0, TE1 // 128)
            def _(blk):
                base = wid * TE1 + blk * 128
                pltpu.sync_copy(src_hbm.at[pl.ds(base, 128)], srcv)
                pltpu.sync_copy(dst_hbm.at[pl.ds(base, 128)], dstv)
                for v in range(8):
                    s16 = srcv[pl.ds(v * 16, 16)]
                    d16 = dstv[pl.ds(v * 16, 16)]
                    a = plsc.load_gather(as_v, [s16]) + \
                        plsc.load_gather(ad_v, [d16])
                    wv = jnp.exp(jnp.where(a >= 0, a, 0.2 * a))
                    gidx = lax.iota(I32, 16) + (base + v * 16)
                    wbuf[pl.ds(v * 16, 16)] = jnp.where(gidx < E, wv, 0.0)
                pltpu.sync_copy(wbuf, w_hbm.at[pl.ds(head * EPAD + base, 128)])
                pltpu.sync_copy(wbuf, dsh.at[dstv], add=True)

            plsc.subcore_barrier()
            pltpu.sync_copy(
                dsh.at[pl.ds(sub * NT, NT)],
                dp_hbm.at[pl.ds((core * 2 + head) * NPAD + sub * NT, NT)])
            plsc.subcore_barrier()

    return _k(asad_t, src, dst)


def k_pass2(hs, src, dst, w):
    @functools.partial(
        pl.kernel, mesh=_mesh(), compiler_params=_SC_PARAMS,
        out_type=jax.ShapeDtypeStruct((4 * NPAD, 32), F32),
        scratch_types=[
            pltpu.VMEM((128,), I32),
            pltpu.VMEM((128,), I32),
            pltpu.VMEM((128,), F32),
            pltpu.VMEM((128, 32), F32),
            pltpu.VMEM_SHARED((NPAD, 32), F32),
            pltpu.SemaphoreType.DMA,
        ],
    )
    def _k(hs_hbm, src_hbm, dst_hbm, w_hbm, acc_hbm,
           srcv, dstv, wvr, gbuf, accsh, sem):
        core = lax.axis_index("c")
        sub = lax.axis_index("s")

        for j in range(2):
            ck = core * 2 + j
            @pl.loop(0, 128)
            def _(r):
                gbuf[r, pl.ds(0, 16)] = jnp.zeros((16,), F32)
                gbuf[r, pl.ds(16, 16)] = jnp.zeros((16,), F32)

            @pl.loop(0, NT // 64)
            def _(t):
                pltpu.sync_copy(gbuf.at[pl.ds(0, 64)],
                                accsh.at[pl.ds(sub * NT + t * 64, 64)])

            plsc.subcore_barrier()

            @pl.loop(0, TE2 // 128)
            def _(blk):
                base = sub * TE2 + blk * 128
                pltpu.sync_copy(src_hbm.at[pl.ds(base, 128)], srcv)
                pltpu.sync_copy(dst_hbm.at[pl.ds(base, 128)], dstv)
                pltpu.sync_copy(w_hbm.at[pl.ds(core * EPAD + base, 128)], wvr)
                for v in range(8):
                    srcv[pl.ds(v * 16, 16)] = \
                        srcv[pl.ds(v * 16, 16)] + ck * NPAD
                pltpu.async_copy(hs_hbm.at[srcv], gbuf, sem).wait()

                @pl.loop(0, 128, step=4)
                def _(e0):
                    for kk in range(4):
                        e = e0 + kk
                        idx16 = lax.broadcast_in_dim(e, (16,), ())
                        wv = plsc.load_gather(wvr, [idx16])
                        gbuf[e, pl.ds(0, 16)] = gbuf[e, pl.ds(0, 16)] * wv
                        gbuf[e, pl.ds(16, 16)] = gbuf[e, pl.ds(16, 16)] * wv

                pltpu.sync_copy(gbuf, accsh.at[dstv], add=True)

            plsc.subcore_barrier()
            pltpu.sync_copy(accsh.at[pl.ds(sub * NT, NT)],
                            acc_hbm.at[pl.ds(ck * NPAD + sub * NT, NT)])
            plsc.subcore_barrier()

    return _k(hs, src, dst, w)


def _chunkify(h):
    return h.reshape(NPAD, 4, 32).transpose(1, 0, 2).reshape(4 * NPAD, 32)


def _unchunkify(a):
    return a.reshape(4, NPAD, 32).transpose(1, 0, 2).reshape(NPAD, 128)


def _gat_layer(x, src, dst, W, a_src, a_dst, b):
    h, asad = k_layer_head(x, W, a_src, a_dst)
    asad_t = asad.T.reshape(4 * NPAD)
    w, dp = k_pass1(asad_t, src, dst)
    acc = k_pass2(_chunkify(h), src, dst, w)
    dp_n = dp.reshape(4, NPAD).T
    return k_combine(_unchunkify(acc), h, asad, dp_n, b)


def kernel(x, edge_index, batch_idx, table, W1, att_src1, att_dst1, b1,
           W2, att_src2, att_dst2, b2, Wp, bp):
    idx = jnp.pad(x[:, 0], (0, NPAD - N))
    src = jnp.pad(edge_index[0], (0, EPAD - E))
    dst = jnp.pad(edge_index[1], (0, EPAD - E))
    bidx = jnp.pad(batch_idx, (0, NPAD - N), constant_values=NG)

    xe = k_embed(table, idx)
    act1 = _gat_layer(xe, src, dst, W1, att_src1, att_dst1, b1)
    act2 = _gat_layer(act1, src, dst, W2, att_src2, att_dst2, b2)
    hp = k_dense(act2, Wp, bp)
    z = k_pool(hp, bidx)
    return (hp[:N], z)


# speedup vs baseline: 67.4953x; 2.1792x over previous
import functools

import jax
import jax.numpy as jnp
from jax import lax
from jax.experimental import pallas as pl
from jax.experimental.pallas import tpu as pltpu
from jax.experimental.pallas import tpu_sc as plsc

N = 50000
E = 800000
NPAD = 50176
EPAD = 802816
NG = 64
F32 = jnp.float32
I32 = jnp.int32

NT = 3136
TE1 = EPAD // 32
TE2 = EPAD // 16

_mesh = lambda: plsc.VectorSubcoreMesh(core_axis_name="c", subcore_axis_name="s",
                                       num_cores=2)
_SC_PARAMS = pltpu.CompilerParams(use_tc_tiling_on_sc=False,
                                  needs_layout_passes=False)


def _leaky(x, s):
    return jnp.where(x >= 0, x, s * x)


def _dense_body(x_ref, w_ref, b_ref, o_ref):
    o_ref[...] = jnp.dot(x_ref[...], w_ref[...],
                         preferred_element_type=F32) + b_ref[...]


def k_dense(x, W, b, blk=3136):
    m = W.shape[1]
    return pl.pallas_call(
        _dense_body,
        grid=(NPAD // blk,),
        in_specs=[
            pl.BlockSpec((blk, x.shape[1]), lambda i: (i, 0)),
            pl.BlockSpec(W.shape, lambda i: (0, 0)),
            pl.BlockSpec((1, m), lambda i: (0, 0)),
        ],
        out_specs=pl.BlockSpec((blk, m), lambda i: (i, 0)),
        out_shape=jax.ShapeDtypeStruct((NPAD, m), F32),
    )(x, W, b.reshape(1, m))


def _layer_head_body(x_ref, w_ref, asrc_ref, adst_ref, h_ref, asad_ref):
    h = jnp.dot(x_ref[...], w_ref[...], preferred_element_type=F32)
    h_ref[...] = h
    a_s, a_d = asrc_ref[...], adst_ref[...]
    cols = []
    for hd in range(2):
        cols.append(jnp.sum(h[:, 64 * hd:64 * hd + 64] * a_s[hd:hd + 1, :],
                            axis=1))
    for hd in range(2):
        cols.append(jnp.sum(h[:, 64 * hd:64 * hd + 64] * a_d[hd:hd + 1, :],
                            axis=1))
    asad_ref[...] = jnp.stack(cols, axis=1)


def k_layer_head(x, W, a_src, a_dst, blk=3136):
    k = x.shape[1]
    return pl.pallas_call(
        _layer_head_body,
        grid=(NPAD // blk,),
        in_specs=[
            pl.BlockSpec((blk, k), lambda i: (i, 0)),
            pl.BlockSpec((k, 128), lambda i: (0, 0)),
            pl.BlockSpec((2, 64), lambda i: (0, 0)),
            pl.BlockSpec((2, 64), lambda i: (0, 0)),
        ],
        out_specs=[
            pl.BlockSpec((blk, 128), lambda i: (i, 0)),
            pl.BlockSpec((blk, 4), lambda i: (i, 0)),
        ],
        out_shape=[
            jax.ShapeDtypeStruct((NPAD, 128), F32),
            jax.ShapeDtypeStruct((NPAD, 4), F32),
        ],
    )(x, W, a_src, a_dst)


def _combine_body(acc_ref, h_ref, asad_ref, dp_ref, b_ref, o_ref):
    asad = asad_ref[...]
    dp = dp_ref[...]
    wself = jnp.exp(_leaky(asad[:, 0:2] + asad[:, 2:4], 0.2))
    denom = dp + wself
    h = h_ref[...]
    acc = acc_ref[...]
    outs = []
    for hd in range(2):
        num = acc[:, 64 * hd:64 * hd + 64] + \
            wself[:, hd:hd + 1] * h[:, 64 * hd:64 * hd + 64]
        outs.append(num / (denom[:, hd:hd + 1] + 1e-16))
    out = jnp.concatenate(outs, axis=1) + b_ref[...]
    o_ref[...] = _leaky(out, 0.01)


def k_combine(accf, h, asad, dp_n, b, blk=3136):
    return pl.pallas_call(
        _combine_body,
        grid=(NPAD // blk,),
        in_specs=[
            pl.BlockSpec((blk, 128), lambda i: (i, 0)),
            pl.BlockSpec((blk, 128), lambda i: (i, 0)),
            pl.BlockSpec((blk, 4), lambda i: (i, 0)),
            pl.BlockSpec((blk, 2), lambda i: (i, 0)),
            pl.BlockSpec((1, 128), lambda i: (0, 0)),
        ],
        out_specs=pl.BlockSpec((blk, 128), lambda i: (i, 0)),
        out_shape=jax.ShapeDtypeStruct((NPAD, 128), F32),
    )(accf, h, asad, dp_n, b.reshape(1, 128))


def _pool_body(hp_ref, bidx_ref, z_ref, sums, cnts):
    i = pl.program_id(0)
    ng = pl.num_programs(0)

    @pl.when(i == 0)
    def _():
        sums[...] = jnp.zeros_like(sums)
        cnts[...] = jnp.zeros_like(cnts)

    bidx = bidx_ref[0]
    giota = lax.broadcasted_iota(I32, (NG, 1), 0)
    oh = (bidx == giota).astype(F32)
    hp = hp_ref[...]
    sums[...] += jnp.dot(oh, hp, preferred_element_type=F32)
    cnts[...] += jnp.dot(oh, jnp.ones_like(hp), preferred_element_type=F32)

    @pl.when(i == ng - 1)
    def _():
        z_ref[...] = sums[...] / jnp.maximum(cnts[...], 1.0)


def k_pool(hp, bidx, blk=3136):
    return pl.pallas_call(
        _pool_body,
        grid=(NPAD // blk,),
        in_specs=[
            pl.BlockSpec((blk, 64), lambda i: (i, 0)),
            pl.BlockSpec((1, 1, blk), lambda i: (i, 0, 0)),
        ],
        out_specs=pl.BlockSpec((NG, 64), lambda i: (0, 0)),
        out_shape=jax.ShapeDtypeStruct((NG, 64), F32),
        scratch_shapes=[pltpu.VMEM((NG, 64), F32), pltpu.VMEM((NG, 64), F32)],
    )(hp, bidx.reshape(NPAD // blk, 1, blk))


def k_embed(table, idx):
    rows_per_tile = NPAD // 32

    @functools.partial(
        pl.kernel, mesh=_mesh(), compiler_params=_SC_PARAMS,
        out_type=jax.ShapeDtypeStruct((NPAD, 64), F32),
        scratch_types=[
            pltpu.VMEM((rows_per_tile,), I32),
            pltpu.VMEM((112, 64), F32),
            pltpu.SemaphoreType.DMA,
        ],
    )
    def _k(table_hbm, idx_hbm, out_hbm, idx_v, rows_v, sem):
        wid = lax.axis_index("c") * 16 + lax.axis_index("s")
        base = wid * rows_per_tile
        pltpu.sync_copy(idx_hbm.at[pl.ds(base, rows_per_tile)], idx_v)

        @pl.loop(0, 14)
        def _(ci):
            off = ci * 112
            pltpu.async_copy(table_hbm.at[idx_v.at[pl.ds(off, 112)]],
                             rows_v, sem).wait()
            pltpu.sync_copy(rows_v, out_hbm.at[pl.ds(base + off, 112)])

    return _k(table, idx)


SLAB_B = 28
SLAB_E = SLAB_B * 128
NSLAB1 = TE1 // SLAB_E
NSLAB2 = TE2 // SLAB_E


def k_pass1(asad_t, src2, dst2):
    @functools.partial(
        pl.kernel, mesh=_mesh(), compiler_params=_SC_PARAMS,
        out_type=[jax.ShapeDtypeStruct((2 * EPAD,), F32),
                  jax.ShapeDtypeStruct((4 * NPAD,), F32)],
        scratch_types=[
            pltpu.VMEM((NPAD,), F32),
            pltpu.VMEM((NPAD,), F32),
            pltpu.VMEM((SLAB_B, 128), I32),
            pltpu.VMEM((SLAB_B, 128), I32),
            pltpu.VMEM((SLAB_E,), F32),
            pltpu.VMEM_SHARED((NPAD,), F32),
            pltpu.SemaphoreType.DMA,
        ],
    )
    def _k(asad_hbm, src_hbm, dst_hbm, w_hbm, dp_hbm,
           as_v, ad_v, srcs, dsts, wsl, dsh, sem):
        core = lax.axis_index("c")
        sub = lax.axis_index("s")
        wid = core * 16 + sub

        for head in range(2):
            @pl.loop(0, NT, step=16)
            def _(i):
                wsl[pl.ds(i, 16)] = jnp.zeros((16,), F32)
            pltpu.sync_copy(wsl.at[pl.ds(0, NT)],
                            dsh.at[pl.ds(sub * NT, NT)])
            pltpu.sync_copy(asad_hbm.at[pl.ds(head * NPAD, NPAD)], as_v)
            pltpu.sync_copy(asad_hbm.at[pl.ds((2 + head) * NPAD, NPAD)], ad_v)
            plsc.subcore_barrier()

            @pl.loop(0, NSLAB1)
            def _(s):
                base = wid * TE1 + s * SLAB_E
                row0 = base // 128
                pltpu.sync_copy(src_hbm.at[pl.ds(row0, SLAB_B)], srcs)
                pltpu.sync_copy(dst_hbm.at[pl.ds(row0, SLAB_B)], dsts)

                @pl.loop(0, SLAB_B)
                def _(b):
                    for v in range(8):
                        s16 = srcs[b, pl.ds(v * 16, 16)]
                        d16 = dsts[b, pl.ds(v * 16, 16)]
                        a = plsc.load_gather(as_v, [s16]) + \
                            plsc.load_gather(ad_v, [d16])
                        wv = jnp.exp(jnp.where(a >= 0, a, 0.2 * a))
                        gidx = lax.iota(I32, 16) + (base + b * 128 + v * 16)
                        wsl[pl.ds(b * 128 + v * 16, 16)] = \
                            jnp.where(gidx < E, wv, 0.0)
                    pltpu.sync_copy(wsl.at[pl.ds(b * 128, 128)],
                                    dsh.at[dsts.at[b]], add=True)

                pltpu.sync_copy(wsl,
                                w_hbm.at[pl.ds(head * EPAD + base, SLAB_E)])

            plsc.subcore_barrier()
            pltpu.sync_copy(
                dsh.at[pl.ds(sub * NT, NT)],
                dp_hbm.at[pl.ds((core * 2 + head) * NPAD + sub * NT, NT)])
            plsc.subcore_barrier()

    return _k(asad_t, src2, dst2)


def k_pass2(hs, src2, dst2, w, j):
    @functools.partial(
        pl.kernel, mesh=_mesh(), compiler_params=_SC_PARAMS,
        out_type=jax.ShapeDtypeStruct((2 * NPAD, 32), F32),
        scratch_types=[
            pltpu.VMEM((SLAB_B, 128), I32),
            pltpu.VMEM((SLAB_B, 128), I32),
            pltpu.VMEM((SLAB_E,), F32),
            pltpu.VMEM((128, 32), F32),
            pltpu.VMEM((128, 32), F32),
            pltpu.VMEM_SHARED((NPAD, 32), F32),
            pltpu.SemaphoreType.DMA,
            pltpu.SemaphoreType.DMA,
        ],
    )
    def _k(hs_hbm, src_hbm, dst_hbm, w_hbm, acc_hbm,
           srcs, dsts, wsl, gb0, gb1, accsh, sem0, sem1):
        core = lax.axis_index("c")
        sub = lax.axis_index("s")
        ckoff = (core * 2 + j) * NPAD

        @pl.loop(0, 128)
        def _(r):
            gb0[r, pl.ds(0, 16)] = jnp.zeros((16,), F32)
            gb0[r, pl.ds(16, 16)] = jnp.zeros((16,), F32)

        @pl.loop(0, NT // 112)
        def _(t):
            pltpu.sync_copy(gb0.at[pl.ds(0, 112)],
                            accsh.at[pl.ds(sub * NT + t * 112, 112)])
        plsc.subcore_barrier()

        @pl.loop(0, NSLAB2)
        def _(s):
            base = sub * TE2 + s * SLAB_E
            row0 = base // 128
            pltpu.sync_copy(src_hbm.at[pl.ds(row0, SLAB_B)], srcs)
            pltpu.sync_copy(dst_hbm.at[pl.ds(row0, SLAB_B)], dsts)
            pltpu.sync_copy(w_hbm.at[pl.ds(core * EPAD + base, SLAB_E)], wsl)

            @pl.loop(0, SLAB_B)
            def _(b):
                for v in range(8):
                    srcs[b, pl.ds(v * 16, 16)] = \
                        srcs[b, pl.ds(v * 16, 16)] + ckoff

            pltpu.async_copy(hs_hbm.at[srcs.at[0]], gb0, sem0)
            pltpu.async_copy(hs_hbm.at[srcs.at[1]], gb1, sem1)

            def _work(bb, gb, sem, start_next):
                pltpu.make_async_copy(hs_hbm.at[srcs.at[bb]],
                                      gb, sem).wait()

                @pl.loop(0, 128, step=8)
                def _(e0):
                    for kk in range(8):
                        e = e0 + kk
                        idx16 = lax.broadcast_in_dim(bb * 128 + e, (16,), ())
                        wv = plsc.load_gather(wsl, [idx16])
                        gb[e, pl.ds(0, 16)] = gb[e, pl.ds(0, 16)] * wv
                        gb[e, pl.ds(16, 16)] = gb[e, pl.ds(16, 16)] * wv

                pltpu.sync_copy(gb, accsh.at[dsts.at[bb]], add=True)
                if start_next:
                    pltpu.async_copy(hs_hbm.at[srcs.at[bb + 2]], gb, sem)

            @pl.loop(0, SLAB_B - 2, step=2)
            def _(b):
                _work(b, gb0, sem0, True)
                _work(b + 1, gb1, sem1, True)

            _work(SLAB_B - 2, gb0, sem0, False)
            _work(SLAB_B - 1, gb1, sem1, False)

        plsc.subcore_barrier()
        pltpu.sync_copy(accsh.at[pl.ds(sub * NT, NT)],
                        acc_hbm.at[pl.ds(core * NPAD + sub * NT, NT)])

    return _k(hs, src2, dst2, w)


def _chunkify(h):
    return h.reshape(NPAD, 4, 32).transpose(1, 0, 2).reshape(4 * NPAD, 32)


def _unchunkify(a):
    return a.reshape(4, NPAD, 32).transpose(1, 0, 2).reshape(NPAD, 128)


def _gat_layer(x, src2, dst2, W, a_src, a_dst, b):
    h, asad = k_layer_head(x, W, a_src, a_dst)
    asad_t = asad.T.reshape(4 * NPAD)
    hs = _chunkify(h)
    w, dp = k_pass1(asad_t, src2, dst2)
    acc_a = k_pass2(hs, src2, dst2, w, 0)
    acc_b = k_pass2(hs, src2, dst2, w, 1)
    a2 = acc_a.reshape(2, NPAD, 32)
    b2 = acc_b.reshape(2, NPAD, 32)
    acc = jnp.stack([a2[0], b2[0], a2[1], b2[1]]).reshape(4 * NPAD, 32)
    dp4 = dp.reshape(4, NPAD)
    dp_n = (dp4[0:2] + dp4[2:4]).T
    return k_combine(_unchunkify(acc), h, asad, dp_n, b)


def kernel(x, edge_index, batch_idx, table, W1, att_src1, att_dst1, b1,
           W2, att_src2, att_dst2, b2, Wp, bp):
    idx = jnp.pad(x[:, 0], (0, NPAD - N))
    src2 = jnp.pad(edge_index[0], (0, EPAD - E)).reshape(EPAD // 128, 128)
    dst2 = jnp.pad(edge_index[1], (0, EPAD - E)).reshape(EPAD // 128, 128)
    bidx = jnp.pad(batch_idx, (0, NPAD - N), constant_values=NG)

    xe = k_embed(table, idx)
    act1 = _gat_layer(xe, src2, dst2, W1, att_src1, att_dst1, b1)
    act2 = _gat_layer(act1, src2, dst2, W2, att_src2, att_dst2, b2)
    hp = k_dense(act2, Wp, bp)
    z = k_pool(hp, bidx)
    return (hp[:N], z)


# speedup vs baseline: 73.3841x; 1.0872x over previous
import functools

import jax
import jax.numpy as jnp
from jax import lax
from jax.experimental import pallas as pl
from jax.experimental.pallas import tpu as pltpu
from jax.experimental.pallas import tpu_sc as plsc

N = 50000
E = 800000
NPAD = 50176
EPAD = 802816
NG = 64
F32 = jnp.float32
I32 = jnp.int32

NT = 3136
TE1 = EPAD // 32
TE2 = EPAD // 16

_mesh = lambda: plsc.VectorSubcoreMesh(core_axis_name="c", subcore_axis_name="s",
                                       num_cores=2)
_SC_PARAMS = pltpu.CompilerParams(use_tc_tiling_on_sc=False,
                                  needs_layout_passes=False)


def _leaky(x, s):
    return jnp.where(x >= 0, x, s * x)


def _dense_body(x_ref, w_ref, b_ref, o_ref):
    o_ref[...] = jnp.dot(x_ref[...], w_ref[...],
                         preferred_element_type=F32) + b_ref[...]


def k_dense(x, W, b, blk=3136):
    m = W.shape[1]
    return pl.pallas_call(
        _dense_body,
        grid=(NPAD // blk,),
        in_specs=[
            pl.BlockSpec((blk, x.shape[1]), lambda i: (i, 0)),
            pl.BlockSpec(W.shape, lambda i: (0, 0)),
            pl.BlockSpec((1, m), lambda i: (0, 0)),
        ],
        out_specs=pl.BlockSpec((blk, m), lambda i: (i, 0)),
        out_shape=jax.ShapeDtypeStruct((NPAD, m), F32),
    )(x, W, b.reshape(1, m))


def _layer_head_body(x_ref, w_ref, asrc_ref, adst_ref, h_ref, asad_ref):
    h = jnp.dot(x_ref[...], w_ref[...], preferred_element_type=F32)
    h_ref[...] = h
    a_s, a_d = asrc_ref[...], adst_ref[...]
    cols = []
    for hd in range(2):
        cols.append(jnp.sum(h[:, 64 * hd:64 * hd + 64] * a_s[hd:hd + 1, :],
                            axis=1))
    for hd in range(2):
        cols.append(jnp.sum(h[:, 64 * hd:64 * hd + 64] * a_d[hd:hd + 1, :],
                            axis=1))
    asad_ref[...] = jnp.stack(cols, axis=1)


def k_layer_head(x, W, a_src, a_dst, blk=3136):
    k = x.shape[1]
    return pl.pallas_call(
        _layer_head_body,
        grid=(NPAD // blk,),
        in_specs=[
            pl.BlockSpec((blk, k), lambda i: (i, 0)),
            pl.BlockSpec((k, 128), lambda i: (0, 0)),
            pl.BlockSpec((2, 64), lambda i: (0, 0)),
            pl.BlockSpec((2, 64), lambda i: (0, 0)),
        ],
        out_specs=[
            pl.BlockSpec((blk, 128), lambda i: (i, 0)),
            pl.BlockSpec((blk, 4), lambda i: (i, 0)),
        ],
        out_shape=[
            jax.ShapeDtypeStruct((NPAD, 128), F32),
            jax.ShapeDtypeStruct((NPAD, 4), F32),
        ],
    )(x, W, a_src, a_dst)


def _combine_body(acc_ref, h_ref, asad_ref, dp_ref, b_ref, o_ref):
    asad = asad_ref[...]
    dp = dp_ref[...]
    wself = jnp.exp(_leaky(asad[:, 0:2] + asad[:, 2:4], 0.2))
    denom = dp + wself
    h = h_ref[...]
    acc = acc_ref[...]
    outs = []
    for hd in range(2):
        num = acc[:, 64 * hd:64 * hd + 64] + \
            wself[:, hd:hd + 1] * h[:, 64 * hd:64 * hd + 64]
        outs.append(num / (denom[:, hd:hd + 1] + 1e-16))
    out = jnp.concatenate(outs, axis=1) + b_ref[...]
    o_ref[...] = _leaky(out, 0.01)


def k_combine(accf, h, asad, dp_n, b, blk=3136):
    return pl.pallas_call(
        _combine_body,
        grid=(NPAD // blk,),
        in_specs=[
            pl.BlockSpec((blk, 128), lambda i: (i, 0)),
            pl.BlockSpec((blk, 128), lambda i: (i, 0)),
            pl.BlockSpec((blk, 4), lambda i: (i, 0)),
            pl.BlockSpec((blk, 2), lambda i: (i, 0)),
            pl.BlockSpec((1, 128), lambda i: (0, 0)),
        ],
        out_specs=pl.BlockSpec((blk, 128), lambda i: (i, 0)),
        out_shape=jax.ShapeDtypeStruct((NPAD, 128), F32),
    )(accf, h, asad, dp_n, b.reshape(1, 128))


def _pool_body(hp_ref, bidx_ref, z_ref, sums, cnts):
    i = pl.program_id(0)
    ng = pl.num_programs(0)

    @pl.when(i == 0)
    def _():
        sums[...] = jnp.zeros_like(sums)
        cnts[...] = jnp.zeros_like(cnts)

    bidx = bidx_ref[0]
    giota = lax.broadcasted_iota(I32, (NG, 1), 0)
    oh = (bidx == giota).astype(F32)
    hp = hp_ref[...]
    sums[...] += jnp.dot(oh, hp, preferred_element_type=F32)
    cnts[...] += jnp.dot(oh, jnp.ones_like(hp), preferred_element_type=F32)

    @pl.when(i == ng - 1)
    def _():
        z_ref[...] = sums[...] / jnp.maximum(cnts[...], 1.0)


def k_pool(hp, bidx, blk=3136):
    return pl.pallas_call(
        _pool_body,
        grid=(NPAD // blk,),
        in_specs=[
            pl.BlockSpec((blk, 64), lambda i: (i, 0)),
            pl.BlockSpec((1, 1, blk), lambda i: (i, 0, 0)),
        ],
        out_specs=pl.BlockSpec((NG, 64), lambda i: (0, 0)),
        out_shape=jax.ShapeDtypeStruct((NG, 64), F32),
        scratch_shapes=[pltpu.VMEM((NG, 64), F32), pltpu.VMEM((NG, 64), F32)],
    )(hp, bidx.reshape(NPAD // blk, 1, blk))


def k_embed(table, idx):
    rows_per_tile = NPAD // 32

    @functools.partial(
        pl.kernel, mesh=_mesh(), compiler_params=_SC_PARAMS,
        out_type=jax.ShapeDtypeStruct((NPAD, 64), F32),
        scratch_types=[
            pltpu.VMEM((rows_per_tile,), I32),
            pltpu.VMEM((112, 64), F32),
            pltpu.SemaphoreType.DMA,
        ],
    )
    def _k(table_hbm, idx_hbm, out_hbm, idx_v, rows_v, sem):
        wid = lax.axis_index("c") * 16 + lax.axis_index("s")
        base = wid * rows_per_tile
        pltpu.sync_copy(idx_hbm.at[pl.ds(base, rows_per_tile)], idx_v)

        @pl.loop(0, 14)
        def _(ci):
            off = ci * 112
            pltpu.async_copy(table_hbm.at[idx_v.at[pl.ds(off, 112)]],
                             rows_v, sem).wait()
            pltpu.sync_copy(rows_v, out_hbm.at[pl.ds(base + off, 112)])

    return _k(table, idx)


SLAB_B = 28
SLAB_E = SLAB_B * 128
NSLAB1 = TE1 // SLAB_E
NSLAB2 = TE2 // SLAB_E


def k_pass1(asad_t, src2, dst2):
    @functools.partial(
        pl.kernel, mesh=_mesh(), compiler_params=_SC_PARAMS,
        out_type=[jax.ShapeDtypeStruct((2 * EPAD,), F32),
                  jax.ShapeDtypeStruct((4 * NPAD,), F32)],
        scratch_types=[
            pltpu.VMEM((NPAD,), F32),
            pltpu.VMEM((NPAD,), F32),
            pltpu.VMEM((SLAB_B, 128), I32),
            pltpu.VMEM((SLAB_B, 128), I32),
            pltpu.VMEM((SLAB_E,), F32),
            pltpu.VMEM_SHARED((NPAD,), F32),
            pltpu.SemaphoreType.DMA,
        ],
    )
    def _k(asad_hbm, src_hbm, dst_hbm, w_hbm, dp_hbm,
           as_v, ad_v, srcs, dsts, wsl, dsh, sem):
        core = lax.axis_index("c")
        sub = lax.axis_index("s")
        wid = core * 16 + sub

        for head in range(2):
            @pl.loop(0, NT, step=16)
            def _(i):
                wsl[pl.ds(i, 16)] = jnp.zeros((16,), F32)
            pltpu.sync_copy(wsl.at[pl.ds(0, NT)],
                            dsh.at[pl.ds(sub * NT, NT)])
            pltpu.sync_copy(asad_hbm.at[pl.ds(head * NPAD, NPAD)], as_v)
            pltpu.sync_copy(asad_hbm.at[pl.ds((2 + head) * NPAD, NPAD)], ad_v)
            plsc.subcore_barrier()

            @pl.loop(0, NSLAB1)
            def _(s):
                base = wid * TE1 + s * SLAB_E
                row0 = base // 128
                pltpu.sync_copy(src_hbm.at[pl.ds(row0, SLAB_B)], srcs)
                pltpu.sync_copy(dst_hbm.at[pl.ds(row0, SLAB_B)], dsts)

                @pl.loop(0, SLAB_B)
                def _(b):
                    for v in range(8):
                        s16 = srcs[b, pl.ds(v * 16, 16)]
                        d16 = dsts[b, pl.ds(v * 16, 16)]
                        a = plsc.load_gather(as_v, [s16]) + \
                            plsc.load_gather(ad_v, [d16])
                        wv = jnp.exp(jnp.where(a >= 0, a, 0.2 * a))
                        gidx = lax.iota(I32, 16) + (base + b * 128 + v * 16)
                        wsl[pl.ds(b * 128 + v * 16, 16)] = \
                            jnp.where(gidx < E, wv, 0.0)
                    pltpu.sync_copy(wsl.at[pl.ds(b * 128, 128)],
                                    dsh.at[dsts.at[b]], add=True)

                pltpu.sync_copy(wsl,
                                w_hbm.at[pl.ds(head * EPAD + base, SLAB_E)])

            plsc.subcore_barrier()
            pltpu.sync_copy(
                dsh.at[pl.ds(sub * NT, NT)],
                dp_hbm.at[pl.ds((core * 2 + head) * NPAD + sub * NT, NT)])
            plsc.subcore_barrier()

    return _k(asad_t, src2, dst2)


def k_pass2(hs, src2, dst2, w, j):
    @functools.partial(
        pl.kernel, mesh=_mesh(), compiler_params=_SC_PARAMS,
        out_type=jax.ShapeDtypeStruct((2 * NPAD, 32), F32),
        scratch_types=[
            pltpu.VMEM((SLAB_B, 128), I32),
            pltpu.VMEM((SLAB_B, 128), I32),
            pltpu.VMEM((SLAB_E,), F32),
            pltpu.VMEM((128, 32), F32),
            pltpu.VMEM((128, 32), F32),
            pltpu.VMEM((128, 32), F32),
            pltpu.VMEM((128, 32), F32),
            pltpu.VMEM_SHARED((NPAD, 32), F32),
            pltpu.SemaphoreType.DMA, pltpu.SemaphoreType.DMA,
            pltpu.SemaphoreType.DMA, pltpu.SemaphoreType.DMA,
            pltpu.SemaphoreType.DMA, pltpu.SemaphoreType.DMA,
            pltpu.SemaphoreType.DMA, pltpu.SemaphoreType.DMA,
        ],
    )
    def _k(hs_hbm, src_hbm, dst_hbm, w_hbm, acc_hbm,
           srcs, dsts, wsl, gb0, gb1, gb2, gb3, accsh,
           sg0, sg1, sg2, sg3, ss0, ss1, ss2, ss3):
        core = lax.axis_index("c")
        sub = lax.axis_index("s")
        ckoff = (core * 2 + j) * NPAD
        gbs = (gb0, gb1, gb2, gb3)
        sgs = (sg0, sg1, sg2, sg3)
        sss = (ss0, ss1, ss2, ss3)

        @pl.loop(0, 128)
        def _(r):
            gb0[r, pl.ds(0, 16)] = jnp.zeros((16,), F32)
            gb0[r, pl.ds(16, 16)] = jnp.zeros((16,), F32)

        @pl.loop(0, NT // 112)
        def _(t):
            pltpu.sync_copy(gb0.at[pl.ds(0, 112)],
                            accsh.at[pl.ds(sub * NT + t * 112, 112)])
        plsc.subcore_barrier()

        def _gather_start(bb, k):
            pltpu.async_copy(hs_hbm.at[srcs.at[bb]], gbs[k], sgs[k])

        def _gather_wait(bb, k):
            pltpu.make_async_copy(hs_hbm.at[srcs.at[bb]],
                                  gbs[k], sgs[k]).wait()

        def _scatter_start(bb, k):
            pltpu.async_copy(gbs[k], accsh.at[dsts.at[bb]], sss[k],
                             add=True)

        def _scatter_wait(bb, k):
            pltpu.make_async_copy(gbs[k], accsh.at[dsts.at[bb]],
                                  sss[k]).wait()

        def _mult(bb, k):
            gb = gbs[k]

            @pl.loop(0, 128, step=8)
            def _(e0):
                for kk in range(8):
                    e = e0 + kk
                    idx16 = lax.broadcast_in_dim(bb * 128 + e, (16,), ())
                    wv = plsc.load_gather(wsl, [idx16])
                    gb[e, pl.ds(0, 16)] = gb[e, pl.ds(0, 16)] * wv
                    gb[e, pl.ds(16, 16)] = gb[e, pl.ds(16, 16)] * wv

        @pl.loop(0, NSLAB2)
        def _(s):
            base = sub * TE2 + s * SLAB_E
            row0 = base // 128
            pltpu.sync_copy(src_hbm.at[pl.ds(row0, SLAB_B)], srcs)
            pltpu.sync_copy(dst_hbm.at[pl.ds(row0, SLAB_B)], dsts)
            pltpu.sync_copy(w_hbm.at[pl.ds(core * EPAD + base, SLAB_E)], wsl)

            @pl.loop(0, SLAB_B)
            def _(b):
                for v in range(8):
                    srcs[b, pl.ds(v * 16, 16)] = \
                        srcs[b, pl.ds(v * 16, 16)] + ckoff

            _gather_start(0, 0)
            _gather_start(1, 1)
            for bb in (0, 1):
                k = bb % 4
                _gather_wait(bb, k)
                _mult(bb, k)
                _scatter_start(bb, k)
                _gather_start(bb + 2, (k + 2) % 4)

            @pl.loop(0, SLAB_B - 4, step=4)
            def _(b):
                for dk in range(4):
                    bb = b + 2 + dk
                    k = (2 + dk) % 4
                    _gather_wait(bb, k)
                    _mult(bb, k)
                    _scatter_start(bb, k)
                    _scatter_wait(bb - 2, (k + 2) % 4)
                    _gather_start(bb + 2, (k + 2) % 4)

            for bb in (SLAB_B - 2, SLAB_B - 1):
                k = bb % 4
                _gather_wait(bb, k)
                _mult(bb, k)
                _scatter_start(bb, k)
                _scatter_wait(bb - 2, (k + 2) % 4)

            _scatter_wait(SLAB_B - 2, (SLAB_B - 2) % 4)
            _scatter_wait(SLAB_B - 1, (SLAB_B - 1) % 4)

        plsc.subcore_barrier()
        pltpu.sync_copy(accsh.at[pl.ds(sub * NT, NT)],
                        acc_hbm.at[pl.ds(core * NPAD + sub * NT, NT)])

    return _k(hs, src2, dst2, w)


def _chunkify(h):
    return h.reshape(NPAD, 4, 32).transpose(1, 0, 2).reshape(4 * NPAD, 32)


def _unchunkify(a):
    return a.reshape(4, NPAD, 32).transpose(1, 0, 2).reshape(NPAD, 128)


def _gat_layer(x, src2, dst2, W, a_src, a_dst, b):
    h, asad = k_layer_head(x, W, a_src, a_dst)
    asad_t = asad.T.reshape(4 * NPAD)
    hs = _chunkify(h)
    w, dp = k_pass1(asad_t, src2, dst2)
    acc_a = k_pass2(hs, src2, dst2, w, 0)
    acc_b = k_pass2(hs, src2, dst2, w, 1)
    a2 = acc_a.reshape(2, NPAD, 32)
    b2 = acc_b.reshape(2, NPAD, 32)
    acc = jnp.stack([a2[0], b2[0], a2[1], b2[1]]).reshape(4 * NPAD, 32)
    dp4 = dp.reshape(4, NPAD)
    dp_n = (dp4[0:2] + dp4[2:4]).T
    return k_combine(_unchunkify(acc), h, asad, dp_n, b)


def kernel(x, edge_index, batch_idx, table, W1, att_src1, att_dst1, b1,
           W2, att_src2, att_dst2, b2, Wp, bp):
    idx = jnp.pad(x[:, 0], (0, NPAD - N))
    src2 = jnp.pad(edge_index[0], (0, EPAD - E)).reshape(EPAD // 128, 128)
    dst2 = jnp.pad(edge_index[1], (0, EPAD - E)).reshape(EPAD // 128, 128)
    bidx = jnp.pad(batch_idx, (0, NPAD - N), constant_values=NG)

    xe = k_embed(table, idx)
    act1 = _gat_layer(xe, src2, dst2, W1, att_src1, att_dst1, b1)
    act2 = _gat_layer(act1, src2, dst2, W2, att_src2, att_dst2, b2)
    hp = k_dense(act2, Wp, bp)
    z = k_pool(hp, bidx)
    return (hp[:N], z)


# speedup vs baseline: 93.8243x; 1.2785x over previous
import functools

import jax
import jax.numpy as jnp
from jax import lax
from jax.experimental import pallas as pl
from jax.experimental.pallas import tpu as pltpu
from jax.experimental.pallas import tpu_sc as plsc

N = 50000
E = 800000
NPAD = 50176
EPAD = 802816
NG = 64
F32 = jnp.float32
I32 = jnp.int32

NT = 3136
TE1 = EPAD // 32
TE2 = EPAD // 16

_mesh = lambda: plsc.VectorSubcoreMesh(core_axis_name="c", subcore_axis_name="s",
                                       num_cores=2)
_SC_PARAMS = pltpu.CompilerParams(use_tc_tiling_on_sc=False,
                                  needs_layout_passes=False)


def _leaky(x, s):
    return jnp.where(x >= 0, x, s * x)


def _dense_body(x_ref, w_ref, b_ref, o_ref):
    o_ref[...] = jnp.dot(x_ref[...], w_ref[...],
                         preferred_element_type=F32) + b_ref[...]


def k_dense(x, W, b, blk=3136):
    m = W.shape[1]
    return pl.pallas_call(
        _dense_body,
        grid=(NPAD // blk,),
        in_specs=[
            pl.BlockSpec((blk, x.shape[1]), lambda i: (i, 0)),
            pl.BlockSpec(W.shape, lambda i: (0, 0)),
            pl.BlockSpec((1, m), lambda i: (0, 0)),
        ],
        out_specs=pl.BlockSpec((blk, m), lambda i: (i, 0)),
        out_shape=jax.ShapeDtypeStruct((NPAD, m), F32),
    )(x, W, b.reshape(1, m))


def _layer_head_body(x_ref, w_ref, asrc_ref, adst_ref, h_ref, asad_ref):
    h = jnp.dot(x_ref[...], w_ref[...], preferred_element_type=F32)
    h_ref[...] = h
    a_s, a_d = asrc_ref[...], adst_ref[...]
    cols = []
    for hd in range(2):
        cols.append(jnp.sum(h[:, 64 * hd:64 * hd + 64] * a_s[hd:hd + 1, :],
                            axis=1))
    for hd in range(2):
        cols.append(jnp.sum(h[:, 64 * hd:64 * hd + 64] * a_d[hd:hd + 1, :],
                            axis=1))
    asad_ref[...] = jnp.stack(cols, axis=1)


def k_layer_head(x, W, a_src, a_dst, blk=3136):
    k = x.shape[1]
    return pl.pallas_call(
        _layer_head_body,
        grid=(NPAD // blk,),
        in_specs=[
            pl.BlockSpec((blk, k), lambda i: (i, 0)),
            pl.BlockSpec((k, 128), lambda i: (0, 0)),
            pl.BlockSpec((2, 64), lambda i: (0, 0)),
            pl.BlockSpec((2, 64), lambda i: (0, 0)),
        ],
        out_specs=[
            pl.BlockSpec((blk, 128), lambda i: (i, 0)),
            pl.BlockSpec((blk, 4), lambda i: (i, 0)),
        ],
        out_shape=[
            jax.ShapeDtypeStruct((NPAD, 128), F32),
            jax.ShapeDtypeStruct((NPAD, 4), F32),
        ],
    )(x, W, a_src, a_dst)


def _combine_body(acc_ref, h_ref, asad_ref, dp_ref, b_ref, o_ref):
    asad = asad_ref[...]
    dp = dp_ref[...]
    wself = jnp.exp(_leaky(asad[:, 0:2] + asad[:, 2:4], 0.2))
    denom = dp + wself
    h = h_ref[...]
    acc = acc_ref[...]
    outs = []
    for hd in range(2):
        num = acc[:, 64 * hd:64 * hd + 64] + \
            wself[:, hd:hd + 1] * h[:, 64 * hd:64 * hd + 64]
        outs.append(num / (denom[:, hd:hd + 1] + 1e-16))
    out = jnp.concatenate(outs, axis=1) + b_ref[...]
    o_ref[...] = _leaky(out, 0.01)


def k_combine(accf, h, asad, dp_n, b, blk=3136):
    return pl.pallas_call(
        _combine_body,
        grid=(NPAD // blk,),
        in_specs=[
            pl.BlockSpec((blk, 128), lambda i: (i, 0)),
            pl.BlockSpec((blk, 128), lambda i: (i, 0)),
            pl.BlockSpec((blk, 4), lambda i: (i, 0)),
            pl.BlockSpec((blk, 2), lambda i: (i, 0)),
            pl.BlockSpec((1, 128), lambda i: (0, 0)),
        ],
        out_specs=pl.BlockSpec((blk, 128), lambda i: (i, 0)),
        out_shape=jax.ShapeDtypeStruct((NPAD, 128), F32),
    )(accf, h, asad, dp_n, b.reshape(1, 128))


def _pool_body(hp_ref, bidx_ref, z_ref, sums, cnts):
    i = pl.program_id(0)
    ng = pl.num_programs(0)

    @pl.when(i == 0)
    def _():
        sums[...] = jnp.zeros_like(sums)
        cnts[...] = jnp.zeros_like(cnts)

    bidx = bidx_ref[0]
    giota = lax.broadcasted_iota(I32, (NG, 1), 0)
    oh = (bidx == giota).astype(F32)
    hp = hp_ref[...]
    sums[...] += jnp.dot(oh, hp, preferred_element_type=F32)
    cnts[...] += jnp.dot(oh, jnp.ones_like(hp), preferred_element_type=F32)

    @pl.when(i == ng - 1)
    def _():
        z_ref[...] = sums[...] / jnp.maximum(cnts[...], 1.0)


def k_pool(hp, bidx, blk=3136):
    return pl.pallas_call(
        _pool_body,
        grid=(NPAD // blk,),
        in_specs=[
            pl.BlockSpec((blk, 64), lambda i: (i, 0)),
            pl.BlockSpec((1, 1, blk), lambda i: (i, 0, 0)),
        ],
        out_specs=pl.BlockSpec((NG, 64), lambda i: (0, 0)),
        out_shape=jax.ShapeDtypeStruct((NG, 64), F32),
        scratch_shapes=[pltpu.VMEM((NG, 64), F32), pltpu.VMEM((NG, 64), F32)],
    )(hp, bidx.reshape(NPAD // blk, 1, blk))


def k_embed(table, idx):
    rows_per_tile = NPAD // 32

    @functools.partial(
        pl.kernel, mesh=_mesh(), compiler_params=_SC_PARAMS,
        out_type=jax.ShapeDtypeStruct((NPAD, 64), F32),
        scratch_types=[
            pltpu.VMEM((rows_per_tile,), I32),
            pltpu.VMEM((112, 64), F32),
            pltpu.SemaphoreType.DMA,
        ],
    )
    def _k(table_hbm, idx_hbm, out_hbm, idx_v, rows_v, sem):
        wid = lax.axis_index("c") * 16 + lax.axis_index("s")
        base = wid * rows_per_tile
        pltpu.sync_copy(idx_hbm.at[pl.ds(base, rows_per_tile)], idx_v)

        @pl.loop(0, 14)
        def _(ci):
            off = ci * 112
            pltpu.async_copy(table_hbm.at[idx_v.at[pl.ds(off, 112)]],
                             rows_v, sem).wait()
            pltpu.sync_copy(rows_v, out_hbm.at[pl.ds(base + off, 112)])

    return _k(table, idx)


SLAB_B = 28
SLAB_E = SLAB_B * 128
NSLAB1 = TE1 // SLAB_E
NSLAB2 = TE2 // SLAB_E


def k_pass1(asad_t, src2, dst2):
    @functools.partial(
        pl.kernel, mesh=_mesh(), compiler_params=_SC_PARAMS,
        out_type=[jax.ShapeDtypeStruct((2 * EPAD,), F32),
                  jax.ShapeDtypeStruct((4 * NPAD,), F32)],
        scratch_types=[
            pltpu.VMEM((NPAD,), F32),
            pltpu.VMEM((NPAD,), F32),
            pltpu.VMEM((SLAB_B, 128), I32),
            pltpu.VMEM((SLAB_B, 128), I32),
            pltpu.VMEM((SLAB_E,), F32),
            pltpu.VMEM_SHARED((NPAD,), F32),
            pltpu.SemaphoreType.DMA,
        ],
    )
    def _k(asad_hbm, src_hbm, dst_hbm, w_hbm, dp_hbm,
           as_v, ad_v, srcs, dsts, wsl, dsh, sem):
        core = lax.axis_index("c")
        sub = lax.axis_index("s")
        wid = core * 16 + sub

        for head in range(2):
            @pl.loop(0, NT, step=16)
            def _(i):
                wsl[pl.ds(i, 16)] = jnp.zeros((16,), F32)
            pltpu.sync_copy(wsl.at[pl.ds(0, NT)],
                            dsh.at[pl.ds(sub * NT, NT)])
            pltpu.sync_copy(asad_hbm.at[pl.ds(head * NPAD, NPAD)], as_v)
            pltpu.sync_copy(asad_hbm.at[pl.ds((2 + head) * NPAD, NPAD)], ad_v)
            plsc.subcore_barrier()

            @pl.loop(0, NSLAB1)
            def _(s):
                base = wid * TE1 + s * SLAB_E
                row0 = base // 128
                pltpu.sync_copy(src_hbm.at[pl.ds(row0, SLAB_B)], srcs)
                pltpu.sync_copy(dst_hbm.at[pl.ds(row0, SLAB_B)], dsts)

                @pl.loop(0, SLAB_B)
                def _(b):
                    for v in range(8):
                        s16 = srcs[b, pl.ds(v * 16, 16)]
                        d16 = dsts[b, pl.ds(v * 16, 16)]
                        a = plsc.load_gather(as_v, [s16]) + \
                            plsc.load_gather(ad_v, [d16])
                        wv = jnp.exp(jnp.where(a >= 0, a, 0.2 * a))
                        gidx = lax.iota(I32, 16) + (base + b * 128 + v * 16)
                        wsl[pl.ds(b * 128 + v * 16, 16)] = \
                            jnp.where(gidx < E, wv, 0.0)
                    pltpu.sync_copy(wsl.at[pl.ds(b * 128, 128)],
                                    dsh.at[dsts.at[b]], add=True)

                pltpu.sync_copy(wsl,
                                w_hbm.at[pl.ds(head * EPAD + base, SLAB_E)])

            plsc.subcore_barrier()
            pltpu.sync_copy(
                dsh.at[pl.ds(sub * NT, NT)],
                dp_hbm.at[pl.ds((core * 2 + head) * NPAD + sub * NT, NT)])
            plsc.subcore_barrier()

    return _k(asad_t, src2, dst2)


def k_pass2(hs, src2, dst2, w):
    @functools.partial(
        pl.kernel, mesh=_mesh(), compiler_params=_SC_PARAMS,
        out_type=jax.ShapeDtypeStruct((4 * NPAD, 32), F32),
        scratch_types=[
            pltpu.VMEM((SLAB_B, 128), I32),
            pltpu.VMEM((SLAB_B, 128), I32),
            pltpu.VMEM((SLAB_E,), F32),
            pltpu.VMEM((128, 32), F32),
            pltpu.VMEM((128, 32), F32),
            pltpu.VMEM((128, 32), F32),
            pltpu.VMEM((128, 32), F32),
            pltpu.VMEM_SHARED((NPAD, 32), F32),
            pltpu.SemaphoreType.DMA, pltpu.SemaphoreType.DMA,
            pltpu.SemaphoreType.DMA, pltpu.SemaphoreType.DMA,
            pltpu.SemaphoreType.DMA, pltpu.SemaphoreType.DMA,
            pltpu.SemaphoreType.DMA, pltpu.SemaphoreType.DMA,
        ],
    )
    def _k(hs_hbm, src_hbm, dst_hbm, w_hbm, acc_hbm,
           srcs, dsts, wsl, gb0, gb1, gb2, gb3, accsh,
           sg0, sg1, sg2, sg3, ss0, ss1, ss2, ss3):
        core = lax.axis_index("c")
        sub = lax.axis_index("s")
        gbs = (gb0, gb1, gb2, gb3)
        sgs = (sg0, sg1, sg2, sg3)
        sss = (ss0, ss1, ss2, ss3)

        def _gather_start(bb, k):
            pltpu.async_copy(hs_hbm.at[srcs.at[bb]], gbs[k], sgs[k])

        def _gather_wait(bb, k):
            pltpu.make_async_copy(hs_hbm.at[srcs.at[bb]],
                                  gbs[k], sgs[k]).wait()

        def _scatter_start(bb, k):
            pltpu.async_copy(gbs[k], accsh.at[dsts.at[bb]], sss[k],
                             add=True)

        def _scatter_wait(bb, k):
            pltpu.make_async_copy(gbs[k], accsh.at[dsts.at[bb]],
                                  sss[k]).wait()

        def _mult(bb, k):
            gb = gbs[k]

            @pl.loop(0, 128, step=16)
            def _(e0):
                wv16 = wsl[pl.ds(bb * 128 + e0, 16)]
                for kk in range(16):
                    e = e0 + kk
                    cidx = jnp.full((16,), kk, I32)
                    wv = wv16.at[cidx].get(mode="promise_in_bounds")
                    gb[e, pl.ds(0, 16)] = gb[e, pl.ds(0, 16)] * wv
                    gb[e, pl.ds(16, 16)] = gb[e, pl.ds(16, 16)] * wv

        def _run_chunk(j, ckoff):
            @pl.loop(0, 128)
            def _(r):
                gb0[r, pl.ds(0, 16)] = jnp.zeros((16,), F32)
                gb0[r, pl.ds(16, 16)] = jnp.zeros((16,), F32)

            @pl.loop(0, NT // 112)
            def _(t):
                pltpu.sync_copy(gb0.at[pl.ds(0, 112)],
                                accsh.at[pl.ds(sub * NT + t * 112, 112)])
            plsc.subcore_barrier()

            @pl.loop(0, NSLAB2)
            def _(s):
                base = sub * TE2 + s * SLAB_E
                row0 = base // 128
                pltpu.sync_copy(src_hbm.at[pl.ds(row0, SLAB_B)], srcs)
                pltpu.sync_copy(dst_hbm.at[pl.ds(row0, SLAB_B)], dsts)
                pltpu.sync_copy(w_hbm.at[pl.ds(core * EPAD + base, SLAB_E)],
                                wsl)

                @pl.loop(0, SLAB_B)
                def _(b):
                    for v in range(8):
                        srcs[b, pl.ds(v * 16, 16)] = \
                            srcs[b, pl.ds(v * 16, 16)] + ckoff

                _gather_start(0, 0)
                _gather_start(1, 1)
                for bb in (0, 1):
                    k = bb % 4
                    _gather_wait(bb, k)
                    _mult(bb, k)
                    _scatter_start(bb, k)
                    _gather_start(bb + 2, (k + 2) % 4)

                @pl.loop(0, SLAB_B - 4, step=4)
                def _(b):
                    for dk in range(4):
                        bb = b + 2 + dk
                        k = (2 + dk) % 4
                        _gather_wait(bb, k)
                        _mult(bb, k)
                        _scatter_start(bb, k)
                        _scatter_wait(bb - 2, (k + 2) % 4)
                        _gather_start(bb + 2, (k + 2) % 4)

                for bb in (SLAB_B - 2, SLAB_B - 1):
                    k = bb % 4
                    _gather_wait(bb, k)
                    _mult(bb, k)
                    _scatter_start(bb, k)
                    _scatter_wait(bb - 2, (k + 2) % 4)

                _scatter_wait(SLAB_B - 2, (SLAB_B - 2) % 4)
                _scatter_wait(SLAB_B - 1, (SLAB_B - 1) % 4)

            plsc.subcore_barrier()
            pltpu.sync_copy(
                accsh.at[pl.ds(sub * NT, NT)],
                acc_hbm.at[pl.ds(ckoff + sub * NT, NT)])
            plsc.subcore_barrier()

        for j in range(2):
            _run_chunk(j, (core * 2 + j) * NPAD)

    return _k(hs, src2, dst2, w)


def _chunkify(h):
    return h.reshape(NPAD, 4, 32).transpose(1, 0, 2).reshape(4 * NPAD, 32)


def _unchunkify(a):
    return a.reshape(4, NPAD, 32).transpose(1, 0, 2).reshape(NPAD, 128)


def _gat_layer(x, src2, dst2, W, a_src, a_dst, b):
    h, asad = k_layer_head(x, W, a_src, a_dst)
    asad_t = asad.T.reshape(4 * NPAD)
    hs = _chunkify(h)
    w, dp = k_pass1(asad_t, src2, dst2)
    acc = k_pass2(hs, src2, dst2, w)
    dp4 = dp.reshape(4, NPAD)
    dp_n = (dp4[0:2] + dp4[2:4]).T
    return k_combine(_unchunkify(acc), h, asad, dp_n, b)


def kernel(x, edge_index, batch_idx, table, W1, att_src1, att_dst1, b1,
           W2, att_src2, att_dst2, b2, Wp, bp):
    idx = jnp.pad(x[:, 0], (0, NPAD - N))
    src2 = jnp.pad(edge_index[0], (0, EPAD - E)).reshape(EPAD // 128, 128)
    dst2 = jnp.pad(edge_index[1], (0, EPAD - E)).reshape(EPAD // 128, 128)
    bidx = jnp.pad(batch_idx, (0, NPAD - N), constant_values=NG)

    xe = k_embed(table, idx)
    act1 = _gat_layer(xe, src2, dst2, W1, att_src1, att_dst1, b1)
    act2 = _gat_layer(act1, src2, dst2, W2, att_src2, att_dst2, b2)
    hp = k_dense(act2, Wp, bp)
    z = k_pool(hp, bidx)
    return (hp[:N], z)
